# baseline (speedup 1.0000x reference)
import functools

import jax
import jax.numpy as jnp
from jax import lax
from jax.experimental import pallas as pl
from jax.experimental.pallas import tpu as pltpu
from jax.experimental.pallas import tpu_sc as plsc

VOCAB = 100000
EMBED = 64
NE = 50
NEP = 128
L = 50
LPAD = 56
B = 4096

NC, NS = 2, 16
NW = NC * NS
CHUNK = 128
NBUF = 2

_PROJ_BLK = 3840


def _proj_body(tableT_ref, w_ref, b_ref, out_ref):
    acc = lax.dot_general(
        tableT_ref[...], w_ref[...],
        dimension_numbers=(((0,), (0,)), ((), ())),
        preferred_element_type=jnp.float32,
    )
    out_ref[...] = acc + b_ref[0:1, :]


def _project_table(tableT, W128, b128):
    return pl.pallas_call(
        _proj_body,
        grid=(pl.cdiv(VOCAB, _PROJ_BLK),),
        in_specs=[
            pl.BlockSpec((EMBED, _PROJ_BLK), lambda i: (0, i)),
            pl.BlockSpec((EMBED, NEP), lambda i: (0, 0)),
            pl.BlockSpec((8, NEP), lambda i: (0, 0)),
        ],
        out_specs=pl.BlockSpec((_PROJ_BLK, NEP), lambda i: (i, 0)),
        out_shape=jax.ShapeDtypeStruct((VOCAB, NEP), jnp.float32),
    )(tableT, W128, b128)


def _gather_body(tp_hbm, idx_hbm, out_hbm, idx_v, bufT, bufs, gsems):
    wid = lax.axis_index("s") * NC + lax.axis_index("c")
    b0 = wid * CHUNK
    pltpu.sync_copy(idx_hbm.at[wid], idx_v)

    def _issue(l, s):
        return pltpu.async_copy(tp_hbm.at[idx_v.at[l]], bufs[s], gsems[s])

    def _wait(s):
        pltpu.make_async_copy(tp_hbm.at[idx_v.at[0]], bufs[s], gsems[s]).wait()

    for s in range(NBUF):
        _issue(s, s)

    iota16 = lax.iota(jnp.int32, 16)

    @pl.loop(0, L, step=NBUF)
    def _l_loop(g):
        for s in range(NBUF):
            l = g + s
            _wait(s)
            buf = bufs[s]
            for e in range(NE):
                col = jnp.full((16,), e, jnp.int32)
                for t8 in range(8):
                    rows = iota16 + (t8 * 16)
                    bufT[e, pl.ds(t8 * 16, 16)] = plsc.load_gather(
                        buf, [rows, col]
                    )
            pltpu.sync_copy(bufT, out_hbm.at[l, :, pl.ds(b0, CHUNK)])
            _issue(jnp.minimum(l + NBUF, L - 1), s)

    for s in range(NBUF):
        _wait(s)


@functools.partial(
    pl.kernel,
    out_type=jax.ShapeDtypeStruct((L, LPAD, B), jnp.float32),
    mesh=plsc.VectorSubcoreMesh(core_axis_name="c", subcore_axis_name="s"),
    scratch_types=[
        pltpu.VMEM((L, CHUNK), jnp.int32),
        pltpu.VMEM((LPAD, CHUNK), jnp.float32),
    ]
    + [pltpu.VMEM((CHUNK, NEP), jnp.float32) for _ in range(NBUF)]
    + [pltpu.SemaphoreType.DMA for _ in range(NBUF)],
    compiler_params=pltpu.CompilerParams(use_tc_tiling_on_sc=False, needs_layout_passes=False),
)
def _sc_gather(tp_hbm, idx_hbm, out_hbm, idx_v, bufT, *rest):
    bufs = rest[:NBUF]
    gsems = rest[NBUF : 2 * NBUF]
    _gather_body(tp_hbm, idx_hbm, out_hbm, idx_v, bufT, bufs, gsems)


def kernel(x, table, W, b):
    W128 = jnp.zeros((EMBED, NEP), jnp.float32).at[:, :NE].set(W)
    b128 = jnp.zeros((8, NEP), jnp.float32).at[:, :NE].set(
        jnp.broadcast_to(b.reshape(1, NE), (8, NE))
    )
    tp = _project_table(jnp.transpose(table), W128, b128)

    xT = jnp.transpose(x.astype(jnp.int32))
    idx = xT.reshape(L, NW, CHUNK).transpose(1, 0, 2)

    y = _sc_gather(tp, idx)
    return jnp.transpose(y, (2, 0, 1))[:, :, :NE]


# speedup vs baseline: 2.0049x; 2.0049x over previous
import functools

import jax
import jax.numpy as jnp
from jax import lax
from jax.experimental import pallas as pl
from jax.experimental.pallas import tpu as pltpu
from jax.experimental.pallas import tpu_sc as plsc

VOCAB = 100000
EMBED = 64
NE = 50
NEP = 128
L = 50
LPAD = 56
B = 4096

NC, NS = 2, 16
NW = NC * NS
N_ROW = B * LPAD
ROWS_PER_W = N_ROW // NW
CHUNK = 128
NCHUNK = ROWS_PER_W // CHUNK
NBUF = 4

_PROJ_BLK = 3840


def _proj_body(tableT_ref, w_ref, b_ref, out_ref):
    acc = lax.dot_general(
        tableT_ref[...], w_ref[...],
        dimension_numbers=(((0,), (0,)), ((), ())),
        preferred_element_type=jnp.float32,
    )
    out_ref[...] = acc + b_ref[0:1, :]


def _project_table(tableT, W128, b128):
    return pl.pallas_call(
        _proj_body,
        grid=(pl.cdiv(VOCAB, _PROJ_BLK),),
        in_specs=[
            pl.BlockSpec((EMBED, _PROJ_BLK), lambda i: (0, i)),
            pl.BlockSpec((EMBED, NEP), lambda i: (0, 0)),
            pl.BlockSpec((8, NEP), lambda i: (0, 0)),
        ],
        out_specs=pl.BlockSpec((_PROJ_BLK, NEP), lambda i: (i, 0)),
        out_shape=jax.ShapeDtypeStruct((VOCAB, NEP), jnp.float32),
    )(tableT, W128, b128)


def _gather_body(tp_hbm, idx_hbm, out_hbm, idx_v, bufs, gsems):
    wid = lax.axis_index("s") * NC + lax.axis_index("c")
    base = wid * ROWS_PER_W
    pltpu.sync_copy(idx_hbm.at[wid], idx_v)

    def _issue(c, b):
        return pltpu.async_copy(tp_hbm.at[idx_v.at[c]], bufs[b], gsems[b])

    def _wait(b):
        pltpu.make_async_copy(tp_hbm.at[idx_v.at[0]], bufs[b], gsems[b]).wait()

    for b in range(NBUF):
        _issue(b, b)

    @pl.loop(0, NCHUNK, step=NBUF)
    def _chunk_loop(g):
        for b in range(NBUF):
            c = g + b
            _wait(b)
            pltpu.sync_copy(bufs[b], out_hbm.at[pl.ds(base + c * CHUNK, CHUNK)])
            _issue(jnp.minimum(c + NBUF, NCHUNK - 1), b)

    for b in range(NBUF):
        _wait(b)


@functools.partial(
    pl.kernel,
    out_type=jax.ShapeDtypeStruct((N_ROW, NEP), jnp.float32),
    mesh=plsc.VectorSubcoreMesh(core_axis_name="c", subcore_axis_name="s"),
    scratch_types=[
        pltpu.VMEM((NCHUNK, CHUNK), jnp.int32),
    ]
    + [pltpu.VMEM((CHUNK, NEP), jnp.float32) for _ in range(NBUF)]
    + [pltpu.SemaphoreType.DMA for _ in range(NBUF)],
    compiler_params=pltpu.CompilerParams(use_tc_tiling_on_sc=False),
)
def _sc_gather(tp_hbm, idx_hbm, out_hbm, idx_v, *rest):
    bufs = rest[:NBUF]
    gsems = rest[NBUF : 2 * NBUF]
    _gather_body(tp_hbm, idx_hbm, out_hbm, idx_v, bufs, gsems)


def kernel(x, table, W, b):
    W128 = jnp.zeros((EMBED, NEP), jnp.float32).at[:, :NE].set(W)
    b128 = jnp.zeros((8, NEP), jnp.float32).at[:, :NE].set(
        jnp.broadcast_to(b.reshape(1, NE), (8, NE))
    )
    tp = _project_table(jnp.transpose(table), W128, b128)

    xi = x.astype(jnp.int32)
    idx = jnp.concatenate([xi, xi[:, : LPAD - L]], axis=1)
    idx = idx.reshape(NW, NCHUNK, CHUNK)

    y = _sc_gather(tp, idx)
    y = y.reshape(B, LPAD, NEP)
    return y[:, :L, :NE]


# speedup vs baseline: 2.1230x; 1.0589x over previous
import functools

import jax
import jax.numpy as jnp
from jax import lax
from jax.experimental import pallas as pl
from jax.experimental.pallas import tpu as pltpu
from jax.experimental.pallas import tpu_sc as plsc

VOCAB = 100000
EMBED = 64
NE = 50
NEP = 128
L = 50
LPAD = 56
B = 4096

NC, NS = 2, 16
NW = NC * NS
N_ROW = B * LPAD
ROWS_PER_W = N_ROW // NW
CHUNK = 128
NCHUNK = ROWS_PER_W // CHUNK
NBUF = 4

_PROJ_BLK = 12800


def _proj_body(tableT_ref, w_ref, b_ref, out_ref):
    acc = lax.dot_general(
        tableT_ref[...], w_ref[...],
        dimension_numbers=(((0,), (0,)), ((), ())),
        preferred_element_type=jnp.float32,
    )
    out_ref[...] = acc + b_ref[0:1, :]


def _project_table(tableT, W128, b128):
    return pl.pallas_call(
        _proj_body,
        grid=(pl.cdiv(VOCAB, _PROJ_BLK),),
        in_specs=[
            pl.BlockSpec((EMBED, _PROJ_BLK), lambda i: (0, i)),
            pl.BlockSpec((EMBED, NEP), lambda i: (0, 0)),
            pl.BlockSpec((8, NEP), lambda i: (0, 0)),
        ],
        out_specs=pl.BlockSpec((_PROJ_BLK, NEP), lambda i: (i, 0)),
        out_shape=jax.ShapeDtypeStruct((VOCAB, NEP), jnp.float32),
    )(tableT, W128, b128)


def _gather_body(tp_hbm, idx_hbm, out_hbm, idx_v, bufs, gsems):
    wid = lax.axis_index("s") * NC + lax.axis_index("c")
    base = wid * ROWS_PER_W
    pltpu.sync_copy(idx_hbm.at[wid], idx_v)

    def _issue(c, b):
        return pltpu.async_copy(tp_hbm.at[idx_v.at[c]], bufs[b], gsems[b])

    def _wait(b):
        pltpu.make_async_copy(tp_hbm.at[idx_v.at[0]], bufs[b], gsems[b]).wait()

    for b in range(NBUF):
        _issue(b, b)

    @pl.loop(0, NCHUNK, step=NBUF)
    def _chunk_loop(g):
        for b in range(NBUF):
            c = g + b
            _wait(b)
            pltpu.sync_copy(bufs[b], out_hbm.at[pl.ds(base + c * CHUNK, CHUNK)])
            _issue(jnp.minimum(c + NBUF, NCHUNK - 1), b)

    for b in range(NBUF):
        _wait(b)


@functools.partial(
    pl.kernel,
    out_type=jax.ShapeDtypeStruct((N_ROW, NEP), jnp.float32),
    mesh=plsc.VectorSubcoreMesh(core_axis_name="c", subcore_axis_name="s"),
    scratch_types=[
        pltpu.VMEM((NCHUNK, CHUNK), jnp.int32),
    ]
    + [pltpu.VMEM((CHUNK, NEP), jnp.float32) for _ in range(NBUF)]
    + [pltpu.SemaphoreType.DMA for _ in range(NBUF)],
    compiler_params=pltpu.CompilerParams(use_tc_tiling_on_sc=False),
)
def _sc_gather(tp_hbm, idx_hbm, out_hbm, idx_v, *rest):
    bufs = rest[:NBUF]
    gsems = rest[NBUF : 2 * NBUF]
    _gather_body(tp_hbm, idx_hbm, out_hbm, idx_v, bufs, gsems)


def kernel(x, table, W, b):
    W128 = jnp.zeros((EMBED, NEP), jnp.float32).at[:, :NE].set(W)
    b128 = jnp.zeros((8, NEP), jnp.float32).at[:, :NE].set(
        jnp.broadcast_to(b.reshape(1, NE), (8, NE))
    )
    tp = _project_table(jnp.transpose(table), W128, b128)

    xi = x.astype(jnp.int32)
    idx = jnp.concatenate([xi, xi[:, : LPAD - L]], axis=1)
    idx = idx.reshape(NW, NCHUNK, CHUNK)

    y = _sc_gather(tp, idx)
    y = y.reshape(B, LPAD, NEP)
    return y[:, :L, :NE]


# speedup vs baseline: 2.1296x; 1.0031x over previous
import functools

import jax
import jax.numpy as jnp
from jax import lax
from jax.experimental import pallas as pl
from jax.experimental.pallas import tpu as pltpu
from jax.experimental.pallas import tpu_sc as plsc

VOCAB = 100000
EMBED = 64
NE = 50
NEP = 128
L = 50
LPAD = 56
B = 4096

NC, NS = 2, 16
NW = NC * NS
N_ROW = B * LPAD
ROWS_PER_W = N_ROW // NW
CHUNK = 128
NCHUNK = ROWS_PER_W // CHUNK
NBUF = 4

_PROJ_BLK = 25600


def _proj_body(tableT_ref, w_ref, b_ref, out_ref):
    acc = lax.dot_general(
        tableT_ref[...], w_ref[...],
        dimension_numbers=(((0,), (0,)), ((), ())),
        preferred_element_type=jnp.float32,
    )
    out_ref[...] = acc + b_ref[0:1, :]


def _project_table(tableT, W128, b128):
    return pl.pallas_call(
        _proj_body,
        grid=(pl.cdiv(VOCAB, _PROJ_BLK),),
        in_specs=[
            pl.BlockSpec((EMBED, _PROJ_BLK), lambda i: (0, i)),
            pl.BlockSpec((EMBED, NEP), lambda i: (0, 0)),
            pl.BlockSpec((8, NEP), lambda i: (0, 0)),
        ],
        out_specs=pl.BlockSpec((_PROJ_BLK, NEP), lambda i: (i, 0)),
        out_shape=jax.ShapeDtypeStruct((VOCAB, NEP), jnp.float32),
    )(tableT, W128, b128)


def _gather_body(tp_hbm, idx_hbm, out_hbm, idx_v, bufs, gsems):
    wid = lax.axis_index("s") * NC + lax.axis_index("c")
    base = wid * ROWS_PER_W
    pltpu.sync_copy(idx_hbm.at[wid], idx_v)

    def _issue(c, b):
        return pltpu.async_copy(tp_hbm.at[idx_v.at[c]], bufs[b], gsems[b])

    def _wait(b):
        pltpu.make_async_copy(tp_hbm.at[idx_v.at[0]], bufs[b], gsems[b]).wait()

    for b in range(NBUF):
        _issue(b, b)

    @pl.loop(0, NCHUNK, step=NBUF)
    def _chunk_loop(g):
        for b in range(NBUF):
            c = g + b
            _wait(b)
            pltpu.sync_copy(bufs[b], out_hbm.at[pl.ds(base + c * CHUNK, CHUNK)])
            _issue(jnp.minimum(c + NBUF, NCHUNK - 1), b)

    for b in range(NBUF):
        _wait(b)


@functools.partial(
    pl.kernel,
    out_type=jax.ShapeDtypeStruct((N_ROW, NEP), jnp.float32),
    mesh=plsc.VectorSubcoreMesh(core_axis_name="c", subcore_axis_name="s"),
    scratch_types=[
        pltpu.VMEM((NCHUNK, CHUNK), jnp.int32),
    ]
    + [pltpu.VMEM((CHUNK, NEP), jnp.float32) for _ in range(NBUF)]
    + [pltpu.SemaphoreType.DMA for _ in range(NBUF)],
    compiler_params=pltpu.CompilerParams(use_tc_tiling_on_sc=False),
)
def _sc_gather(tp_hbm, idx_hbm, out_hbm, idx_v, *rest):
    bufs = rest[:NBUF]
    gsems = rest[NBUF : 2 * NBUF]
    _gather_body(tp_hbm, idx_hbm, out_hbm, idx_v, bufs, gsems)


def kernel(x, table, W, b):
    W128 = jnp.zeros((EMBED, NEP), jnp.float32).at[:, :NE].set(W)
    b128 = jnp.zeros((8, NEP), jnp.float32).at[:, :NE].set(
        jnp.broadcast_to(b.reshape(1, NE), (8, NE))
    )
    tp = _project_table(jnp.transpose(table), W128, b128)

    xi = x.astype(jnp.int32)
    idx = jnp.concatenate([xi, xi[:, : LPAD - L]], axis=1)
    idx = idx.reshape(NW, NCHUNK, CHUNK)

    y = _sc_gather(tp, idx)
    y = y.reshape(B, LPAD, NEP)
    return y[:, :L, :NE]
